# baseline (speedup 1.0000x reference)
import jax
import jax.numpy as jnp
from jax import lax
from jax.experimental import pallas as pl
from jax.experimental.pallas import tpu as pltpu
from jax.experimental.pallas import tpu_sc as plsc

_B = 64
_P = 1024
_D = 768
_NC = 2
_NS = 16
_L = 16
_NW = _NC * _NS
_ROWS = _P // _NW
_NBUF = 2
_BB = 2
_UNROLL = 8


def _sc_body(x_hbm, t_hbm, o_hbm, tbuf, b0, b1, *sems):
    bufs = (b0, b1)
    lsems = sems[:_NBUF]
    ssems = sems[_NBUF:]
    wid = lax.axis_index("s") * _NC + lax.axis_index("c")
    r0 = wid * _ROWS

    def start_load(j, k):
        pltpu.async_copy(
            x_hbm.at[pl.ds(_BB * j, _BB), pl.ds(r0, _ROWS), :], bufs[k], lsems[k]
        )

    def wait_load(k):
        pltpu.make_async_copy(
            x_hbm.at[pl.ds(0, _BB), pl.ds(0, _ROWS), :], bufs[k], lsems[k]
        ).wait()

    def start_store(j, k):
        pltpu.async_copy(
            bufs[k], o_hbm.at[pl.ds(_BB * j, _BB), pl.ds(r0, _ROWS), :], ssems[k]
        )

    def wait_store(k):
        pltpu.make_async_copy(
            bufs[k], o_hbm.at[pl.ds(0, _BB), pl.ds(0, _ROWS), :], ssems[k]
        ).wait()

    for k in range(_NBUF - 1):
        start_load(k, k)
    pltpu.sync_copy(t_hbm.at[pl.ds(r0, _ROWS), :], tbuf)

    def add_chunk(k):
        buf = bufs[k]

        for u in range(_BB):
            @plsc.parallel_loop(0, _ROWS, step=1)
            def _(r):
                @plsc.parallel_loop(0, _D, step=_L, unroll=_UNROLL)
                def _(c):
                    sl = pl.ds(c, _L)
                    plsc.addupdate(buf.at[u, r, sl], tbuf[r, sl])

    _NJ = _B // _BB

    def group(g, _):
        for k in range(_NBUF):
            j = g * _NBUF + k
            ka = (k + _NBUF - 1) % _NBUF

            wait_load(k)
            add_chunk(k)

            @pl.when(j + _NBUF - 1 < _NJ)
            def _():
                @pl.when(j >= 1)
                def _():
                    wait_store(ka)

                start_load(j + _NBUF - 1, ka)

            start_store(j, k)
        return 0

    lax.fori_loop(0, _NJ // _NBUF, group, 0, unroll=False)

    for k in range(_NBUF):
        wait_store(k)


def kernel(x, table):
    mesh = plsc.VectorSubcoreMesh(core_axis_name="c", subcore_axis_name="s")
    scratch = [pltpu.VMEM((_ROWS, _D), jnp.float32)]
    scratch += [pltpu.VMEM((_BB, _ROWS, _D), jnp.float32) for _ in range(_NBUF)]
    scratch += [pltpu.SemaphoreType.DMA] * (2 * _NBUF)
    run = pl.kernel(
        _sc_body,
        mesh=mesh,
        out_type=jax.ShapeDtypeStruct((_B, _P, _D), jnp.float32),
        scratch_types=scratch,
        compiler_params=pltpu.CompilerParams(use_tc_tiling_on_sc=True),
    )
    return run(x, table)


# speedup vs baseline: 1.6605x; 1.6605x over previous
import jax
import jax.numpy as jnp
from jax import lax
from jax.experimental import pallas as pl
from jax.experimental.pallas import tpu as pltpu
from jax.experimental.pallas import tpu_sc as plsc

_B = 64
_P = 1024
_D = 768
_NC = 2
_NS = 16
_L = 16
_NW = _NC * _NS
_ROWS = _P // _NW
_NBUF = 4
_UNROLL = 8


def _sc_body(x_hbm, t_hbm, o_hbm, tbuf, b0, b1, b2, b3, *sems):
    bufs = (b0, b1, b2, b3)
    lsems = sems[:_NBUF]
    ssems = sems[_NBUF:]
    wid = lax.axis_index("s") * _NC + lax.axis_index("c")
    r0 = wid * _ROWS

    def start_load(b, k):
        pltpu.async_copy(x_hbm.at[b, pl.ds(r0, _ROWS), :], bufs[k], lsems[k])

    def wait_load(k):
        pltpu.make_async_copy(
            x_hbm.at[0, pl.ds(0, _ROWS), :], bufs[k], lsems[k]
        ).wait()

    def start_store(b, k):
        pltpu.async_copy(bufs[k], o_hbm.at[b, pl.ds(r0, _ROWS), :], ssems[k])

    def wait_store(k):
        pltpu.make_async_copy(
            bufs[k], o_hbm.at[0, pl.ds(0, _ROWS), :], ssems[k]
        ).wait()

    for k in range(_NBUF - 1):
        start_load(k, k)
    pltpu.sync_copy(t_hbm.at[pl.ds(r0, _ROWS), :], tbuf)

    def add_chunk(k):
        buf = bufs[k]

        @plsc.parallel_loop(0, _ROWS, step=1)
        def _(r):
            @plsc.parallel_loop(0, _D, step=_L, unroll=_UNROLL)
            def _(c):
                sl = pl.ds(c, _L)
                plsc.addupdate(buf.at[r, sl], tbuf[r, sl])

    def group(g, _):
        for k in range(_NBUF):
            b = g * _NBUF + k
            ka = (k + _NBUF - 1) % _NBUF

            wait_load(k)
            add_chunk(k)
            start_store(b, k)

            @pl.when(b + _NBUF - 1 < _B)
            def _():
                @pl.when(b >= 1)
                def _():
                    wait_store(ka)

                start_load(b + _NBUF - 1, ka)
        return 0

    lax.fori_loop(0, _B // _NBUF, group, 0, unroll=False)

    for k in range(_NBUF):
        wait_store(k)


def kernel(x, table):
    mesh = plsc.VectorSubcoreMesh(core_axis_name="c", subcore_axis_name="s")
    scratch = [pltpu.VMEM((_ROWS, _D), jnp.float32) for _ in range(1 + _NBUF)]
    scratch += [pltpu.SemaphoreType.DMA] * (2 * _NBUF)
    run = pl.kernel(
        _sc_body,
        mesh=mesh,
        out_type=jax.ShapeDtypeStruct((_B, _P, _D), jnp.float32),
        scratch_types=scratch,
        compiler_params=pltpu.CompilerParams(use_tc_tiling_on_sc=True),
    )
    return run(x, table)
